# baseline (speedup 1.0000x reference)
import numpy as np

import jax
import jax.numpy as jnp
from jax.experimental import pallas as pl
from jax.experimental.pallas import tpu as pltpu

_BF16 = jnp.bfloat16
_F32 = jnp.float32


def _cnn_body(x_ref, ws_ref, bs_ref, wf1_ref, bf1_ref, wf2_ref, bf2_ref,
              wf3_ref, bf3_ref, out_ref, p0_ref, p1_ref, p2_ref):
    def dot(a, b):
        return jnp.dot(a, b, preferred_element_type=_F32)

    p0_ref[...] = x_ref[...].astype(_BF16)

    w1 = ws_ref[0:192, :]
    b1 = bs_ref[0:96, :]
    for jj in range(14):
        x0 = p0_ref[pl.ds(jj * 192, 480), :]
        x1 = p0_ref[pl.ds(jj * 192 + 96, 480), :]
        y0 = dot(w1, x0)
        y1 = dot(w1, x1)
        y = jnp.maximum(jnp.maximum(y0[0:96], y0[96:192]),
                        jnp.maximum(y1[0:96], y1[96:192]))
        y = jnp.maximum(y + b1, 0.0)
        p1_ref[pl.ds(jj * 96, 96), :] = y.astype(_BF16)

    w2 = ws_ref[192:352, :]
    b2 = bs_ref[96:176, :]
    for jj in range(5):
        x0 = p1_ref[pl.ds(jj * 192, 480), :]
        x1 = p1_ref[pl.ds(jj * 192 + 96, 480), :]
        y0 = dot(w2, x0)
        y1 = dot(w2, x1)
        y = jnp.maximum(jnp.maximum(y0[0:80], y0[80:160]),
                        jnp.maximum(y1[0:80], y1[80:160]))
        y = jnp.maximum(y + b2, 0.0)
        p2_ref[pl.ds(jj * 80, 80), :] = y.astype(_BF16)

    h1 = jnp.maximum(dot(wf1_ref[...], p2_ref[...]) + bf1_ref[...], 0.0)
    h2 = jnp.maximum(dot(wf2_ref[...], h1) + bf2_ref[...], 0.0)
    logits = dot(wf3_ref[...], h2) + bf3_ref[...]

    m = jnp.max(logits, axis=0, keepdims=True)
    z = logits - m
    s = jnp.sum(jnp.exp(z), axis=0, keepdims=True)
    out_ref[...] = z - jnp.log(s)


def _toeplitz_onehot(n_u, n_w):
    e = np.zeros((5, 2, n_u, n_w), np.float32)
    for k in range(5):
        for p in range(2):
            for u in range(n_u):
                e[k, p, u, 2 * u + p + k] = 1.0
    return jnp.asarray(e.reshape(5, 2 * n_u * n_w))


def kernel(x, w1s, b1s, w2s, b2s, wf1, bf1, wf2, bf2, wf3, bf3):
    B = x.shape[0]
    TB = 1024 if B >= 1024 else B
    Bp = ((B + TB - 1) // TB) * TB

    xr = jnp.transpose(x, (2, 1, 3, 0)).reshape(3072, B)
    if Bp != B:
        xr = jnp.pad(xr, ((0, 0), (0, Bp - B)))

    t1 = jnp.dot(w1s.reshape(90, 5), _toeplitz_onehot(14, 32))
    t1 = t1.reshape(6, 3, 5, 2, 14, 32).transpose(3, 0, 4, 2, 1, 5)
    m1 = jnp.pad(t1.reshape(2, 84, 480), ((0, 0), (0, 12), (0, 0)))
    t2 = jnp.dot(w2s.reshape(480, 5), _toeplitz_onehot(5, 14))
    t2 = t2.reshape(16, 6, 5, 2, 5, 14).transpose(3, 0, 4, 2, 1, 5)
    m2 = jnp.pad(t2.reshape(2, 80, 5, 84), ((0, 0), (0, 0), (0, 0), (0, 12)))
    ws = jnp.concatenate([m1.reshape(192, 480),
                          m2.reshape(160, 480)]).astype(_BF16)

    bs = jnp.concatenate([jnp.pad(jnp.repeat(b1s, 14), (0, 12)),
                          jnp.repeat(b2s, 5)]).reshape(176, 1)

    wf1f = jnp.transpose(wf1, (1, 0, 2)).reshape(120, 400).astype(_BF16)

    def vfull(a):
        return pl.BlockSpec(a.shape, lambda i: (0,) * a.ndim)

    in_specs = [
        pl.BlockSpec((3072, TB), lambda i: (0, i)),
        vfull(ws), vfull(bs),
        vfull(wf1f), vfull(bf1), vfull(wf2), vfull(bf2), vfull(wf3),
        vfull(bf3),
    ]

    macs = Bp * (6 * 28 * 28 * 75 + 16 * 10 * 10 * 150
                 + 400 * 120 + 120 * 84 + 84 * 10)
    cost = pl.CostEstimate(flops=2 * macs,
                           transcendentals=11 * Bp,
                           bytes_accessed=4 * Bp * 3072 + 4 * Bp * 10 + 70000)

    out = pl.pallas_call(
        _cnn_body,
        out_shape=jax.ShapeDtypeStruct((10, Bp), _F32),
        grid=(Bp // TB,),
        in_specs=in_specs,
        out_specs=pl.BlockSpec((10, TB), lambda i: (0, i)),
        scratch_shapes=[pltpu.VMEM((3072, TB), _BF16),
                        pltpu.VMEM((14 * 96, TB), _BF16),
                        pltpu.VMEM((5 * 80, TB), _BF16)],
        compiler_params=pltpu.CompilerParams(
            dimension_semantics=("parallel",),
            vmem_limit_bytes=48 * 1024 * 1024),
        cost_estimate=cost,
    )(xr, ws, bs, wf1f, bf1, wf2, bf2, wf3, bf3)

    return jnp.transpose(out)[:B]


# speedup vs baseline: 1.0143x; 1.0143x over previous
import numpy as np

import jax
import jax.numpy as jnp
from jax.experimental import pallas as pl
from jax.experimental.pallas import tpu as pltpu

_BF16 = jnp.bfloat16
_F32 = jnp.float32


def _cnn_body(x_ref, ws_ref, bs_ref, wf1_ref, bf1_ref, wf2_ref, bf2_ref,
              wf3_ref, bf3_ref, out_ref, p0_ref, p1_ref, p2_ref):
    def dot(a, b):
        return jnp.dot(a, b, preferred_element_type=_F32)

    p0_ref[...] = x_ref[...].astype(_BF16)

    w1 = ws_ref[0:192, :]
    b1 = bs_ref[0:96, :]
    for jj in range(14):
        x0 = p0_ref[pl.ds(jj * 192, 480), :]
        x1 = p0_ref[pl.ds(jj * 192 + 96, 480), :]
        y0 = dot(w1, x0)
        y1 = dot(w1, x1)
        y = jnp.maximum(jnp.maximum(y0[0:96], y0[96:192]),
                        jnp.maximum(y1[0:96], y1[96:192]))
        y = jnp.maximum(y + b1, 0.0)
        p1_ref[pl.ds(jj * 96, 96), :] = y.astype(_BF16)

    w2 = ws_ref[192:352, :]
    b2 = bs_ref[96:176, :]
    for jj in range(5):
        x0 = p1_ref[pl.ds(jj * 192, 480), :]
        x1 = p1_ref[pl.ds(jj * 192 + 96, 480), :]
        y0 = dot(w2, x0)
        y1 = dot(w2, x1)
        y = jnp.maximum(jnp.maximum(y0[0:80], y0[80:160]),
                        jnp.maximum(y1[0:80], y1[80:160]))
        y = jnp.maximum(y + b2, 0.0)
        p2_ref[pl.ds(jj * 80, 80), :] = y.astype(_BF16)

    h1 = jnp.maximum(dot(wf1_ref[...], p2_ref[...]) + bf1_ref[...], 0.0)
    h2 = jnp.maximum(dot(wf2_ref[...], h1) + bf2_ref[...], 0.0)
    logits = dot(wf3_ref[...], h2) + bf3_ref[...]

    m = jnp.max(logits, axis=0, keepdims=True)
    z = logits - m
    s = jnp.sum(jnp.exp(z), axis=0, keepdims=True)
    out_ref[...] = z - jnp.log(s)


def _toeplitz_onehot(n_u, n_w):
    e = np.zeros((5, 2, n_u, n_w), np.float32)
    for k in range(5):
        for p in range(2):
            for u in range(n_u):
                e[k, p, u, 2 * u + p + k] = 1.0
    return jnp.asarray(e.reshape(5, 2 * n_u * n_w))


def kernel(x, w1s, b1s, w2s, b2s, wf1, bf1, wf2, bf2, wf3, bf3):
    B = x.shape[0]
    TB = 512 if B >= 512 else B
    Bp = ((B + TB - 1) // TB) * TB

    xr = jnp.transpose(x, (2, 1, 3, 0)).reshape(3072, B)
    if Bp != B:
        xr = jnp.pad(xr, ((0, 0), (0, Bp - B)))

    t1 = jnp.dot(w1s.reshape(90, 5), _toeplitz_onehot(14, 32))
    t1 = t1.reshape(6, 3, 5, 2, 14, 32).transpose(3, 0, 4, 2, 1, 5)
    m1 = jnp.pad(t1.reshape(2, 84, 480), ((0, 0), (0, 12), (0, 0)))
    t2 = jnp.dot(w2s.reshape(480, 5), _toeplitz_onehot(5, 14))
    t2 = t2.reshape(16, 6, 5, 2, 5, 14).transpose(3, 0, 4, 2, 1, 5)
    m2 = jnp.pad(t2.reshape(2, 80, 5, 84), ((0, 0), (0, 0), (0, 0), (0, 12)))
    ws = jnp.concatenate([m1.reshape(192, 480),
                          m2.reshape(160, 480)]).astype(_BF16)

    bs = jnp.concatenate([jnp.pad(jnp.repeat(b1s, 14), (0, 12)),
                          jnp.repeat(b2s, 5)]).reshape(176, 1)

    wf1f = jnp.transpose(wf1, (1, 0, 2)).reshape(120, 400).astype(_BF16)

    def vfull(a):
        return pl.BlockSpec(a.shape, lambda i: (0,) * a.ndim)

    in_specs = [
        pl.BlockSpec((3072, TB), lambda i: (0, i)),
        vfull(ws), vfull(bs),
        vfull(wf1f), vfull(bf1), vfull(wf2), vfull(bf2), vfull(wf3),
        vfull(bf3),
    ]

    macs = Bp * (6 * 28 * 28 * 75 + 16 * 10 * 10 * 150
                 + 400 * 120 + 120 * 84 + 84 * 10)
    cost = pl.CostEstimate(flops=2 * macs,
                           transcendentals=11 * Bp,
                           bytes_accessed=4 * Bp * 3072 + 4 * Bp * 10 + 70000)

    out = pl.pallas_call(
        _cnn_body,
        out_shape=jax.ShapeDtypeStruct((10, Bp), _F32),
        grid=(Bp // TB,),
        in_specs=in_specs,
        out_specs=pl.BlockSpec((10, TB), lambda i: (0, i)),
        scratch_shapes=[pltpu.VMEM((3072, TB), _BF16),
                        pltpu.VMEM((14 * 96, TB), _BF16),
                        pltpu.VMEM((5 * 80, TB), _BF16)],
        compiler_params=pltpu.CompilerParams(
            dimension_semantics=("parallel",),
            vmem_limit_bytes=48 * 1024 * 1024),
        cost_estimate=cost,
    )(xr, ws, bs, wf1f, bf1, wf2, bf2, wf3, bf3)

    return jnp.transpose(out)[:B]
